# baseline (speedup 1.0000x reference)
import functools

import jax
import jax.numpy as jnp
from jax import lax
from jax.experimental import pallas as pl
from jax.experimental.pallas import tpu as pltpu
from jax.experimental.pallas import tpu_sc as plsc

VOCAB = 100000
D = 128
B, M, S = 1024, 200, 4
NC, NS = 2, 16
NW = NC * NS
IPG = 128
NSPLIT = 4
BS = B // NSPLIT
POS_SL = BS * M
POS_W = POS_SL // NW
GPP = 64
RPP = GPP * S
STEPS = POS_W // GPP
IDX_ROWS = POS_W * S // IPG


def _sc_gather_pool(idx3, C0, C1, C2, C3):
    mesh = plsc.VectorSubcoreMesh(core_axis_name="c", subcore_axis_name="s")
    out = jax.ShapeDtypeStruct((POS_SL, D), jnp.float32)

    @functools.partial(
        pl.kernel,
        mesh=mesh,
        out_type=[out, out, out, out],
        scratch_types=[
            pltpu.VMEM((IDX_ROWS, IPG), jnp.int32),
            pltpu.VMEM((RPP, D), jnp.float32),
            pltpu.VMEM((RPP, D), jnp.float32),
            pltpu.VMEM((GPP, D), jnp.float32),
            pltpu.VMEM((GPP, D), jnp.float32),
            pltpu.SemaphoreType.DMA,
            pltpu.SemaphoreType.DMA,
            pltpu.SemaphoreType.DMA,
            pltpu.SemaphoreType.DMA,
        ],
    )
    def k(idx_hbm, t0, t1, t2, t3, e0, e1, e2, e3,
          idx_v, raw0, raw1, pool0, pool1, gsem0, gsem1, osem0, osem1):
        wid = lax.axis_index("s") * NC + lax.axis_index("c")
        base = wid * POS_W
        pltpu.sync_copy(idx_hbm.at[wid], idx_v)

        def fire_gather(table, s, raw, gsem):
            pltpu.async_copy(table.at[idx_v.at[2 * s]],
                             raw.at[pl.ds(0, IPG)], gsem)
            pltpu.async_copy(table.at[idx_v.at[2 * s + 1]],
                             raw.at[pl.ds(IPG, IPG)], gsem)

        def wait_gather(table, s, raw, gsem):
            pltpu.make_async_copy(table.at[idx_v.at[2 * s]],
                                  raw.at[pl.ds(0, IPG)], gsem).wait()
            pltpu.make_async_copy(table.at[idx_v.at[2 * s + 1]],
                                  raw.at[pl.ds(IPG, IPG)], gsem).wait()

        def fire_out(e_out, s, pool, osem):
            pltpu.async_copy(pool, e_out.at[pl.ds(base + s * GPP, GPP)], osem)

        def wait_out(e_out, s, pool, osem):
            pltpu.make_async_copy(
                pool, e_out.at[pl.ds(base + s * GPP, GPP)], osem).wait()

        def pool(raw, pool_v):
            @plsc.parallel_loop(0, GPP, unroll=8)
            def _(i):
                r = i * S
                for j in range(D // 16):
                    sl = pl.ds(j * 16, 16)
                    pool_v[i, sl] = ((raw[r, sl] + raw[r + 1, sl])
                                     + (raw[r + 2, sl] + raw[r + 3, sl]))

        for table, e_out in ((t0, e0), (t1, e1), (t2, e2), (t3, e3)):
            def step_pair(su, _, table=table, e_out=e_out):
                a = 2 * su
                b = a + 1
                fire_gather(table, b, raw1, gsem1)
                wait_gather(table, a, raw0, gsem0)

                @pl.when(a >= 2)
                def _():
                    wait_out(e_out, a - 2, pool0, osem0)
                pool(raw0, pool0)
                fire_out(e_out, a, pool0, osem0)

                @pl.when(a + 2 < STEPS)
                def _():
                    fire_gather(table, a + 2, raw0, gsem0)
                wait_gather(table, b, raw1, gsem1)

                @pl.when(b >= 2)
                def _():
                    wait_out(e_out, b - 2, pool1, osem1)
                pool(raw1, pool1)
                fire_out(e_out, b, pool1, osem1)
                return 0

            fire_gather(table, 0, raw0, gsem0)
            lax.fori_loop(0, STEPS // 2, step_pair, 0)
            if STEPS % 2 == 1:
                s_last = STEPS - 1
                wait_gather(table, s_last, raw0, gsem0)
                wait_out(e_out, s_last - 2, pool0, osem0)
                pool(raw0, pool0)
                fire_out(e_out, s_last, pool0, osem0)
                wait_out(e_out, s_last - 1, pool1, osem1)
                wait_out(e_out, s_last, pool0, osem0)
            else:
                wait_out(e_out, STEPS - 2, pool0, osem0)
                wait_out(e_out, STEPS - 1, pool1, osem1)

    return k(idx3, C0, C1, C2, C3)


BB = 32


def _hops_body(e0, e1, e2, e3, h, lg_out, u_out):
    u = h[:, 0, :]
    tables = (e0, e1, e2, e3)
    logit = None
    for hop in range(3):
        eh = tables[hop][...]
        logit = jnp.sum(eh * u[:, None, :], axis=2)
        p = jax.nn.softmax(logit, axis=1)
        en = tables[hop + 1][...]
        o = jnp.sum(en * p[:, :, None], axis=1)
        u = u + o
    lg_out[...] = logit
    u_out[...] = u


def _tc_hops(E0, E1, E2, E3, hidden):
    espec = pl.BlockSpec((BB, M, D), lambda i: (i, 0, 0))
    return pl.pallas_call(
        _hops_body,
        grid=(BS // BB,),
        in_specs=[espec, espec, espec, espec,
                  pl.BlockSpec((BB, 1, D), lambda i: (i, 0, 0))],
        out_specs=[pl.BlockSpec((BB, M), lambda i: (i, 0)),
                   pl.BlockSpec((BB, D), lambda i: (i, 0))],
        out_shape=[jax.ShapeDtypeStruct((BS, M), jnp.float32),
                   jax.ShapeDtypeStruct((BS, D), jnp.float32)],
    )(E0, E1, E2, E3, hidden)


def kernel(story, hidden, C0, C1, C2, C3):
    logits, us = [], []
    for k in range(NSPLIT):
        sl = slice(k * BS, (k + 1) * BS)
        idx3 = story[sl].reshape(NW, IDX_ROWS, IPG)
        E0, E1, E2, E3 = _sc_gather_pool(idx3, C0, C1, C2, C3)
        lg, u = _tc_hops(E0.reshape(BS, M, D), E1.reshape(BS, M, D),
                         E2.reshape(BS, M, D), E3.reshape(BS, M, D),
                         hidden[sl])
        logits.append(lg)
        us.append(u)
    return (jnp.concatenate(logits, axis=0), jnp.concatenate(us, axis=0))


# speedup vs baseline: 1.0267x; 1.0267x over previous
import functools

import jax
import jax.numpy as jnp
from jax import lax
from jax.experimental import pallas as pl
from jax.experimental.pallas import tpu as pltpu
from jax.experimental.pallas import tpu_sc as plsc

VOCAB = 100000
D = 128
B, M, S = 1024, 200, 4
NC, NS = 2, 16
NW = NC * NS
IPG = 128
NSPLIT = 4
BS = B // NSPLIT
POS_SL = BS * M
POS_W = POS_SL // NW
GPP = 64
RPP = GPP * S
STEPS = POS_W // GPP
IDX_ROWS = POS_W * S // IPG


def _sc_gather_pool(idx3, C0, C1, C2, C3):
    mesh = plsc.VectorSubcoreMesh(core_axis_name="c", subcore_axis_name="s")
    out = jax.ShapeDtypeStruct((POS_SL, D), jnp.float32)

    @functools.partial(
        pl.kernel,
        mesh=mesh,
        out_type=[out, out, out, out],
        scratch_types=[
            pltpu.VMEM((IDX_ROWS, IPG), jnp.int32),
            pltpu.VMEM((RPP, D), jnp.float32),
            pltpu.VMEM((RPP, D), jnp.float32),
            pltpu.VMEM((RPP, D), jnp.float32),
            pltpu.VMEM((GPP, D), jnp.float32),
            pltpu.VMEM((GPP, D), jnp.float32),
            pltpu.VMEM((GPP, D), jnp.float32),
            pltpu.SemaphoreType.DMA,
            pltpu.SemaphoreType.DMA,
            pltpu.SemaphoreType.DMA,
            pltpu.SemaphoreType.DMA,
            pltpu.SemaphoreType.DMA,
            pltpu.SemaphoreType.DMA,
        ],
    )
    def k(idx_hbm, t0, t1, t2, t3, e0, e1, e2, e3,
          idx_v, raw0, raw1, raw2, pool0, pool1, pool2,
          gsem0, gsem1, gsem2, osem0, osem1, osem2):
        wid = lax.axis_index("s") * NC + lax.axis_index("c")
        base = wid * POS_W
        pltpu.sync_copy(idx_hbm.at[wid], idx_v)

        def fire_gather(table, s, raw, gsem):
            pltpu.async_copy(table.at[idx_v.at[2 * s]],
                             raw.at[pl.ds(0, IPG)], gsem)
            pltpu.async_copy(table.at[idx_v.at[2 * s + 1]],
                             raw.at[pl.ds(IPG, IPG)], gsem)

        def wait_gather(table, s, raw, gsem):
            pltpu.make_async_copy(table.at[idx_v.at[2 * s]],
                                  raw.at[pl.ds(0, IPG)], gsem).wait()
            pltpu.make_async_copy(table.at[idx_v.at[2 * s + 1]],
                                  raw.at[pl.ds(IPG, IPG)], gsem).wait()

        def fire_out(e_out, s, pool, osem):
            pltpu.async_copy(pool, e_out.at[pl.ds(base + s * GPP, GPP)], osem)

        def wait_out(e_out, s, pool, osem):
            pltpu.make_async_copy(
                pool, e_out.at[pl.ds(base + s * GPP, GPP)], osem).wait()

        def pool(raw, pool_v):
            @plsc.parallel_loop(0, GPP, unroll=4)
            def _(i):
                r = i * S
                for j in range(D // 16):
                    sl = pl.ds(j * 16, 16)
                    pool_v[i, sl] = ((raw[r, sl] + raw[r + 1, sl])
                                     + (raw[r + 2, sl] + raw[r + 3, sl]))

        for table, e_out in ((t0, e0), (t1, e1), (t2, e2), (t3, e3)):
            bufs = ((raw0, pool0, gsem0, osem0),
                    (raw1, pool1, gsem1, osem1),
                    (raw2, pool2, gsem2, osem2))

            def triple(su, _, table=table, e_out=e_out):
                a = 3 * su
                for kk, (rawk, poolk, gsemk, osemk) in enumerate(bufs):
                    s = a + kk
                    wait_gather(table, s, rawk, gsemk)

                    @pl.when(s >= 3)
                    def _(s=s, poolk=poolk, osemk=osemk):
                        wait_out(e_out, s - 3, poolk, osemk)
                    pool(rawk, poolk)
                    fire_out(e_out, s, poolk, osemk)

                    @pl.when(s + 3 < STEPS)
                    def _(s=s, rawk=rawk, gsemk=gsemk):
                        fire_gather(table, s + 3, rawk, gsemk)
                return 0

            for kk, (rawk, _pk, gsemk, _ok) in enumerate(bufs):
                fire_gather(table, kk, rawk, gsemk)
            lax.fori_loop(0, STEPS // 3, triple, 0)
            for s_tail in range((STEPS // 3) * 3, STEPS):
                rawk, poolk, gsemk, osemk = bufs[s_tail % 3]
                wait_gather(table, s_tail, rawk, gsemk)
                wait_out(e_out, s_tail - 3, poolk, osemk)
                pool(rawk, poolk)
                fire_out(e_out, s_tail, poolk, osemk)
            for s_drain in range(STEPS - 3, STEPS):
                _rk, poolk, _gk, osemk = bufs[s_drain % 3]
                wait_out(e_out, s_drain, poolk, osemk)

    return k(idx3, C0, C1, C2, C3)


BB = 32


def _hops_body(e0, e1, e2, e3, h, lg_out, u_out):
    u = h[:, 0, :]
    tables = (e0, e1, e2, e3)
    logit = None
    for hop in range(3):
        eh = tables[hop][...]
        logit = jnp.sum(eh * u[:, None, :], axis=2)
        p = jax.nn.softmax(logit, axis=1)
        en = tables[hop + 1][...]
        o = jnp.sum(en * p[:, :, None], axis=1)
        u = u + o
    lg_out[...] = logit
    u_out[...] = u


def _tc_hops(E0, E1, E2, E3, hidden):
    espec = pl.BlockSpec((BB, M, D), lambda i: (i, 0, 0))
    return pl.pallas_call(
        _hops_body,
        grid=(BS // BB,),
        in_specs=[espec, espec, espec, espec,
                  pl.BlockSpec((BB, 1, D), lambda i: (i, 0, 0))],
        out_specs=[pl.BlockSpec((BB, M), lambda i: (i, 0)),
                   pl.BlockSpec((BB, D), lambda i: (i, 0))],
        out_shape=[jax.ShapeDtypeStruct((BS, M), jnp.float32),
                   jax.ShapeDtypeStruct((BS, D), jnp.float32)],
    )(E0, E1, E2, E3, hidden)


def kernel(story, hidden, C0, C1, C2, C3):
    logits, us = [], []
    for k in range(NSPLIT):
        sl = slice(k * BS, (k + 1) * BS)
        idx3 = story[sl].reshape(NW, IDX_ROWS, IPG)
        E0, E1, E2, E3 = _sc_gather_pool(idx3, C0, C1, C2, C3)
        lg, u = _tc_hops(E0.reshape(BS, M, D), E1.reshape(BS, M, D),
                         E2.reshape(BS, M, D), E3.reshape(BS, M, D),
                         hidden[sl])
        logits.append(lg)
        us.append(u)
    return (jnp.concatenate(logits, axis=0), jnp.concatenate(us, axis=0))
